# Initial kernel scaffold; baseline (speedup 1.0000x reference)
import jax
import jax.numpy as jnp
from jax.experimental import pallas as pl


def kernel(inputs, imf, weights):
    raise NotImplementedError("write your pallas kernel here")


# speedup vs baseline: 2.7522x; 2.7522x over previous
import jax
import jax.numpy as jnp
from jax.experimental import pallas as pl
from jax.experimental.pallas import tpu as pltpu

B, N = 64, 1024


def _scale_kernel(hours_ref, perm_ref, w_ref, imf_ref, out_ref):
    b = pl.program_id(0)
    h = hours_ref[b]
    wv = jnp.maximum(w_ref[h, 0], 0.0)
    out_ref[...] = imf_ref[...] * wv


def kernel(inputs, imf, weights):
    hours = (inputs[:, 0, 0, 1] * 24.0).astype(jnp.int32)
    perm = jnp.argsort(hours)
    hours_sorted = jnp.take(hours, perm, axis=0)

    grid_spec = pltpu.PrefetchScalarGridSpec(
        num_scalar_prefetch=2,
        grid=(B,),
        in_specs=[
            pl.BlockSpec((24, 1), lambda b, hr, pr: (0, 0),
                         memory_space=pltpu.SMEM),
            pl.BlockSpec((1, N, N), lambda b, hr, pr: (hr[b], 0, 0)),
        ],
        out_specs=pl.BlockSpec((1, N, N), lambda b, hr, pr: (pr[b], 0, 0)),
    )

    out = pl.pallas_call(
        _scale_kernel,
        grid_spec=grid_spec,
        out_shape=jax.ShapeDtypeStruct((B, N, N), jnp.float32),
    )(hours_sorted, perm, weights, imf)
    return out
